# Initial kernel scaffold; baseline (speedup 1.0000x reference)
import jax
import jax.numpy as jnp
from jax.experimental import pallas as pl


def kernel(item_features, user_features, item_nids, edge_src_0, edge_dst_0, edge_src_1, edge_dst_1, edge_src_2, edge_dst_2, edge_src_3, edge_dst_3, edge_src_4, edge_dst_4, item_id_table, W_rev_0, b_rev_0, W_rev_1, b_rev_1, W_rev_2, b_rev_2, W_rev_3, b_rev_3, W_rev_4, b_rev_4, W_agg, b_agg):
    raise NotImplementedError("write your pallas kernel here")


# speedup vs baseline: 1.3539x; 1.3539x over previous
import functools
import jax
import jax.numpy as jnp
from jax.experimental import pallas as pl
from jax.experimental.pallas import tpu as pltpu

R = 5
DIN = 128
D = 64


def _transform_body(x_ref, wrev_ref, brev_ref, wagg_ref, out_ref):
    x = x_ref[...]
    for r in range(R):
        m = jnp.dot(x, wrev_ref[r], preferred_element_type=jnp.float32) + brev_ref[r]
        p = jnp.dot(m, wagg_ref[pl.ds(D * (r + 1), D), :],
                    preferred_element_type=jnp.float32)
        out_ref[r] = p


def _final_body(uf_ref, s_ref, cnt_ref, wagg_ref, bagg_ref, out_ref):
    acc = jnp.dot(uf_ref[...], wagg_ref[pl.ds(0, D), :],
                  preferred_element_type=jnp.float32)
    for r in range(R):
        inv = 1.0 / jnp.maximum(cnt_ref[r], 1.0)
        acc = acc + s_ref[r] * inv
    acc = acc + bagg_ref[...]
    out_ref[...] = jnp.where(acc >= 0, acc, 0.01 * acc)


def _item_transform(x, wrev, brev, wagg, block=2000):
    n = x.shape[0]
    grid = (n // block,)
    return pl.pallas_call(
        _transform_body,
        grid=grid,
        in_specs=[
            pl.BlockSpec((block, DIN), lambda i: (i, 0)),
            pl.BlockSpec((R, DIN, D), lambda i: (0, 0, 0)),
            pl.BlockSpec((R, D), lambda i: (0, 0)),
            pl.BlockSpec((D * (R + 1), D), lambda i: (0, 0)),
        ],
        out_specs=pl.BlockSpec((R, block, D), lambda i: (0, i, 0)),
        out_shape=jax.ShapeDtypeStruct((R, n, D), jnp.float32),
    )(x, wrev, brev, wagg)


def _final(uf, s, cnt, wagg, bagg, block=2000):
    u = uf.shape[0]
    grid = (u // block,)
    return pl.pallas_call(
        _final_body,
        grid=grid,
        in_specs=[
            pl.BlockSpec((block, D), lambda i: (i, 0)),
            pl.BlockSpec((R, block, D), lambda i: (0, i, 0)),
            pl.BlockSpec((R, block, 1), lambda i: (0, i, 0)),
            pl.BlockSpec((D * (R + 1), D), lambda i: (0, 0)),
            pl.BlockSpec((1, D), lambda i: (0, 0)),
        ],
        out_specs=pl.BlockSpec((block, D), lambda i: (i, 0)),
        out_shape=jax.ShapeDtypeStruct((u, D), jnp.float32),
    )(uf, s, cnt, wagg, bagg)


def kernel(item_features, user_features, item_nids,
           edge_src_0, edge_dst_0, edge_src_1, edge_dst_1,
           edge_src_2, edge_dst_2, edge_src_3, edge_dst_3,
           edge_src_4, edge_dst_4,
           item_id_table,
           W_rev_0, b_rev_0, W_rev_1, b_rev_1, W_rev_2, b_rev_2,
           W_rev_3, b_rev_3, W_rev_4, b_rev_4,
           W_agg, b_agg):
    n_users = user_features.shape[0]
    item_id_emb = jnp.take(item_id_table, item_nids, axis=0)
    x = jnp.concatenate([item_features, item_id_emb], axis=1)
    wrev = jnp.stack([W_rev_0, W_rev_1, W_rev_2, W_rev_3, W_rev_4])
    brev = jnp.stack([b_rev_0, b_rev_1, b_rev_2, b_rev_3, b_rev_4])

    p = _item_transform(x, wrev, brev, W_agg)

    srcs = [edge_src_0, edge_src_1, edge_src_2, edge_src_3, edge_src_4]
    dsts = [edge_dst_0, edge_dst_1, edge_dst_2, edge_dst_3, edge_dst_4]
    s_list = []
    cnt_list = []
    for r in range(R):
        rows = jnp.take(p[r], srcs[r], axis=0)
        s_list.append(jnp.zeros((n_users, D), jnp.float32).at[dsts[r]].add(rows))
        cnt_list.append(jnp.zeros((n_users,), jnp.float32).at[dsts[r]].add(1.0))
    s = jnp.stack(s_list)
    cnt = jnp.stack(cnt_list)[:, :, None]

    return _final(user_features, s, cnt, W_agg, b_agg.reshape(1, D))
